# Initial kernel scaffold; baseline (speedup 1.0000x reference)
import jax
import jax.numpy as jnp
from jax.experimental import pallas as pl


def kernel(x, W1, b1, W2, b2, edge_index):
    raise NotImplementedError("write your pallas kernel here")


# speedup vs baseline: 3.1244x; 3.1244x over previous
import functools

import jax
import jax.numpy as jnp
from jax import lax
from jax.experimental import pallas as pl
from jax.experimental.pallas import tpu as pltpu
from jax.experimental.pallas import tpu_sc as plsc

N = 10000
F = 128
E = 320000
K_PROP = 10
ALPHA = 0.1

NC = 2
NS = 16
NPAD = 10240
RPT = NPAD // NS
RSUB = 128
NSUB = RPT // RSUB
CHUNK = 128
EPT = E // NS
NCH = 162
FP = 16
NPASS = (F // NC) // FP
G = NC * NPASS

_mesh = plsc.VectorSubcoreMesh(core_axis_name="cc", subcore_axis_name="ss")


def _mlp_body(x_ref, w1_ref, b1_ref, w2_ref, b2_ref, o_ref):
    h1 = jnp.dot(x_ref[...], w1_ref[...], preferred_element_type=jnp.float32)
    h1 = jnp.maximum(h1 + b1_ref[...], 0.0)
    o_ref[...] = (
        jnp.dot(h1, w2_ref[...], preferred_element_type=jnp.float32)
        + b2_ref[...]
    )


_BM = 512


def _mlp(x_pad, W1, b1, W2, b2):
    grid = (NPAD // _BM,)
    return pl.pallas_call(
        _mlp_body,
        grid=grid,
        in_specs=[
            pl.BlockSpec((_BM, F), lambda i: (i, 0)),
            pl.BlockSpec((F, F), lambda i: (0, 0)),
            pl.BlockSpec((1, F), lambda i: (0, 0)),
            pl.BlockSpec((F, F), lambda i: (0, 0)),
            pl.BlockSpec((1, F), lambda i: (0, 0)),
        ],
        out_specs=pl.BlockSpec((_BM, F), lambda i: (i, 0)),
        out_shape=jax.ShapeDtypeStruct((NPAD, F), jnp.float32),
    )(x_pad, W1, b1, W2, b2)


def _deg_body(dst_hbm, deg_out, dst_v, ones_v, tmp_v, deg_sh):
    cid = lax.axis_index("cc")
    sid = lax.axis_index("ss")
    base = sid * RPT
    pltpu.sync_copy(dst_hbm.at[sid], dst_v)

    def fill_ones(i, carry):
        ones_v[pl.ds(i * 16, 16)] = jnp.full((16,), 1.0, jnp.float32)
        return carry

    lax.fori_loop(0, CHUNK // 16, fill_ones, 0)

    def fill_zero(i, carry):
        tmp_v[pl.ds(i * 16, 16)] = jnp.zeros((16,), jnp.float32)
        return carry

    lax.fori_loop(0, RPT // 16, fill_zero, 0)
    pltpu.sync_copy(tmp_v, deg_sh.at[pl.ds(base, RPT)])
    plsc.subcore_barrier()

    half = NCH // NC

    def scat(j, carry):
        pltpu.sync_copy(ones_v, deg_sh.at[dst_v.at[cid * half + j]], add=True)
        return carry

    lax.fori_loop(0, half, scat, 0)
    plsc.subcore_barrier()
    pltpu.sync_copy(deg_sh.at[pl.ds(base, RPT)], tmp_v)
    pltpu.sync_copy(tmp_v, deg_out.at[cid, pl.ds(base, RPT)])


_deg_call = functools.partial(
    pl.kernel,
    out_type=jax.ShapeDtypeStruct((NC, NPAD), jnp.float32),
    mesh=_mesh,
    compiler_params=pltpu.CompilerParams(use_tc_tiling_on_sc=False),
    scratch_types=[
        pltpu.VMEM((NCH, CHUNK), jnp.int32),
        pltpu.VMEM((CHUNK,), jnp.float32),
        pltpu.VMEM((RPT,), jnp.float32),
        pltpu.VMEM_SHARED((NPAD,), jnp.float32),
    ],
)(_deg_body)


def _prep_body(deg_ref, h_ref, y0_ref, c_ref, dis_ref):
    pid = pl.program_id(0)
    rows = lax.broadcasted_iota(jnp.int32, (_BM, 1), 0) + pid * _BM
    valid = rows < N
    degp = deg_ref[:, 0:1] + deg_ref[:, 1:2] + 1.0
    dis = lax.rsqrt(degp)
    c = 1.0 / degp
    dis = jnp.where(valid, dis, 0.0)
    c = jnp.where(valid, c, 0.0)
    y0_ref[...] = dis * h_ref[...]
    c_ref[...] = c
    dis_ref[...] = dis


def _prep(deg2t, h_pad):
    grid = (NPAD // _BM,)
    return pl.pallas_call(
        _prep_body,
        grid=grid,
        in_specs=[
            pl.BlockSpec((_BM, 2), lambda i: (i, 0)),
            pl.BlockSpec((_BM, F), lambda i: (i, 0)),
        ],
        out_specs=[
            pl.BlockSpec((_BM, F), lambda i: (i, 0)),
            pl.BlockSpec((_BM, 1), lambda i: (i, 0)),
            pl.BlockSpec((_BM, 1), lambda i: (i, 0)),
        ],
        out_shape=[
            jax.ShapeDtypeStruct((NPAD, F), jnp.float32),
            jax.ShapeDtypeStruct((NPAD, 1), jnp.float32),
            jax.ShapeDtypeStruct((NPAD, 1), jnp.float32),
        ],
    )(deg2t, h_pad)


def _prop_body(
    y0_hbm, c_hbm, src_hbm, dst_hbm, y_hbm,
    src_v, dst_v, gbuf, ybuf, sbuf, zbuf, g_v, c_v, s_sh,
):
    cid = lax.axis_index("cc")
    sid = lax.axis_index("ss")
    base = sid * RPT

    pltpu.sync_copy(src_hbm.at[sid], src_v)
    pltpu.sync_copy(dst_hbm.at[sid], dst_v)
    pltpu.sync_copy(c_hbm.at[pl.ds(base, RPT)], c_v)

    for p in range(NPASS):
        slab = cid * NPASS + p
        pltpu.sync_copy(y0_hbm.at[slab, pl.ds(base, RPT)], g_v.at[p])
        pltpu.sync_copy(g_v.at[p], y_hbm.at[pl.ds(slab * NPAD + base, RPT)])

    def g_scale(r, carry):
        for p in range(NPASS):
            g_v[p, r, :] = g_v[p, r, :] * ALPHA
        return carry

    lax.fori_loop(0, RPT, g_scale, 0)

    def z_fill(r, carry):
        zbuf[r, :] = jnp.zeros((FP,), jnp.float32)
        return carry

    lax.fori_loop(0, RSUB, z_fill, 0)

    def s_zero(i, carry):
        pltpu.sync_copy(zbuf, s_sh.at[pl.ds(base + i * RSUB, RSUB)])
        return carry

    for p in range(NPASS):
        slab = cid * NPASS + p
        goff = slab * NPAD
        delta = (cid * NPASS) * NPAD if p == 0 else NPAD

        def adj(b, carry):
            row = b // (CHUNK // 16)
            col = (b % (CHUNK // 16)) * 16
            sl = pl.ds(col, 16)
            src_v[row, sl] = src_v[row, sl] + delta
            return carry

        lax.fori_loop(0, NCH * (CHUNK // 16), adj, 0)

        lax.fori_loop(0, NSUB, s_zero, 0)
        plsc.subcore_barrier()

        def step(k, carry):
            def edge(j, c2):
                pltpu.sync_copy(y_hbm.at[src_v.at[j]], gbuf)
                pltpu.sync_copy(gbuf, s_sh.at[dst_v.at[j]], add=True)
                return c2

            lax.fori_loop(0, NCH, edge, 0)
            plsc.subcore_barrier()

            def dense(i, c2):
                srows = pl.ds(base + i * RSUB, RSUB)
                yrows = pl.ds(goff + base + i * RSUB, RSUB)
                pltpu.sync_copy(y_hbm.at[yrows], ybuf)
                pltpu.sync_copy(s_sh.at[srows], sbuf)

                def dgroup(rb, c3):
                    c16 = c_v[pl.ds(i * RSUB + rb * 16, 16)] * (1.0 - ALPHA)
                    for r2 in range(16):
                        r = rb * 16 + r2
                        gr = i * RSUB + r
                        cs = c16[r2]
                        ybuf[r, :] = (
                            (sbuf[r, :] + ybuf[r, :]) * cs + g_v[p, gr, :]
                        )
                    return c3

                lax.fori_loop(0, RSUB // 16, dgroup, 0)
                pltpu.sync_copy(ybuf, y_hbm.at[yrows])
                pltpu.sync_copy(zbuf, s_sh.at[srows])
                return c2

            lax.fori_loop(0, NSUB, dense, 0)
            plsc.subcore_barrier()
            return carry

        lax.fori_loop(0, K_PROP, step, 0)


_prop_call = functools.partial(
    pl.kernel,
    out_type=jax.ShapeDtypeStruct((G * NPAD, FP), jnp.float32),
    mesh=_mesh,
    compiler_params=pltpu.CompilerParams(use_tc_tiling_on_sc=False),
    scratch_types=[
        pltpu.VMEM((NCH, CHUNK), jnp.int32),
        pltpu.VMEM((NCH, CHUNK), jnp.int32),
        pltpu.VMEM((CHUNK, FP), jnp.float32),
        pltpu.VMEM((RSUB, FP), jnp.float32),
        pltpu.VMEM((RSUB, FP), jnp.float32),
        pltpu.VMEM((RSUB, FP), jnp.float32),
        pltpu.VMEM((NPASS, RPT, FP), jnp.float32),
        pltpu.VMEM((RPT,), jnp.float32),
        pltpu.VMEM_SHARED((NPAD, FP), jnp.float32),
    ],
)(_prop_body)


_BO = 400


def _out_body(y_ref, dis_ref, o_ref):
    z = y_ref[...] / dis_ref[...]
    m = jnp.max(z, axis=1, keepdims=True)
    zc = z - m
    ssum = jnp.sum(jnp.exp(zc), axis=1, keepdims=True)
    o_ref[...] = zc - jnp.log(ssum)


def _final(y_full, dis_n):
    grid = (N // _BO,)
    return pl.pallas_call(
        _out_body,
        grid=grid,
        in_specs=[
            pl.BlockSpec((_BO, F), lambda i: (i, 0)),
            pl.BlockSpec((_BO, 1), lambda i: (i, 0)),
        ],
        out_specs=pl.BlockSpec((_BO, F), lambda i: (i, 0)),
        out_shape=jax.ShapeDtypeStruct((N, F), jnp.float32),
    )(y_full, dis_n)


def kernel(x, W1, b1, W2, b2, edge_index):
    src = edge_index[0].astype(jnp.int32)
    dst = edge_index[1].astype(jnp.int32)
    pad_e = NCH * CHUNK - EPT
    srcc = jnp.pad(
        src.reshape(NS, EPT), ((0, 0), (0, pad_e)), constant_values=0
    ).reshape(NS, NCH, CHUNK)
    dstc = jnp.pad(
        dst.reshape(NS, EPT), ((0, 0), (0, pad_e)), constant_values=NPAD - 1
    ).reshape(NS, NCH, CHUNK)

    x_pad = jnp.pad(x, ((0, NPAD - N), (0, 0)))
    h_pad = _mlp(x_pad, W1, b1.reshape(1, F), W2, b2.reshape(1, F))

    deg2 = _deg_call(dstc)
    deg2t = deg2.T

    y0, c, dis = _prep(deg2t, h_pad)
    y0g = y0.reshape(NPAD, G, FP).transpose(1, 0, 2)

    yK = _prop_call(y0g, c.reshape(NPAD), srcc, dstc)

    y_full = (
        yK.reshape(G, NPAD, FP).transpose(1, 0, 2).reshape(NPAD, F)[:N]
    )
    return _final(y_full, dis[:N])
